# baseline (speedup 1.0000x reference)
import functools

import jax
import jax.numpy as jnp
from jax import lax
from jax.experimental import pallas as pl
from jax.experimental.pallas import tpu as pltpu
from jax.experimental.pallas import tpu_sc as plsc

_NC, _NSUB = 2, 16
_NW = _NC * _NSUB
_NSEG = 128
_OPAD = 64


def _sc_gather(table, idx):
    bb, ll = idx.shape
    d = table.shape[1]
    rows_per_w = (bb * ll) // _NW
    w_per_row = ll // rows_per_w
    mesh = plsc.VectorSubcoreMesh(core_axis_name="c", subcore_axis_name="s")
    nstream = 4
    chunk = rows_per_w // nstream

    @functools.partial(
        pl.kernel,
        mesh=mesh,
        out_type=jax.ShapeDtypeStruct((bb, ll, d), jnp.float32),
        scratch_types=(
            [pltpu.VMEM((rows_per_w,), jnp.int32),
             pltpu.VMEM((rows_per_w, d), jnp.float32)]
            + [pltpu.SemaphoreType.DMA] * (2 * nstream)
        ),
    )
    def gather_kernel(table_hbm, idx_hbm, out_hbm, idx_v, rows_v, *sems):
        wid = lax.axis_index("s") * _NC + lax.axis_index("c")
        row = wid // w_per_row
        col = (wid % w_per_row) * rows_per_w
        pltpu.sync_copy(idx_hbm.at[row, pl.ds(col, rows_per_w)], idx_v)
        gathers = []
        for i in range(nstream):
            gathers.append(pltpu.async_copy(
                table_hbm.at[idx_v.at[pl.ds(i * chunk, chunk)]],
                rows_v.at[pl.ds(i * chunk, chunk)], sems[i]))
        writes = []
        for i in range(nstream):
            gathers[i].wait()
            writes.append(pltpu.async_copy(
                rows_v.at[pl.ds(i * chunk, chunk)],
                out_hbm.at[row, pl.ds(col + i * chunk, chunk)],
                sems[nstream + i]))
        for wcp in writes:
            wcp.wait()

    return gather_kernel(table, idx)


def _tc_body(x_ref, w_ref, b_ref, w2_ref, wid_ref, am_ref, bc_ref, o_ref,
             *, maxS, maxT, rows_per_step):
    ll = wid_ref.shape[-1]
    hh = x_ref.shape[-1]
    x = x_ref[...].reshape(rows_per_step * ll, hh).astype(jnp.bfloat16)
    h = jnp.tanh(
        jnp.dot(x, w_ref[...], preferred_element_type=jnp.float32) + b_ref[...]
    )
    scores = lax.dot_general(
        h.astype(jnp.bfloat16), w2_ref[...],
        (((1,), (1,)), ((), ())), preferred_element_type=jnp.float32,
    )
    lt = (
        lax.broadcasted_iota(jnp.int32, (ll, ll), 0)
        <= lax.broadcasted_iota(jnp.int32, (ll, ll), 1)
    ).astype(jnp.bfloat16)
    sid = lax.broadcasted_iota(jnp.int32, (_NSEG, ll), 0)
    e1 = (lax.broadcasted_iota(jnp.int32, (1, 2), 1) == 1).astype(jnp.float32)
    for r in range(rows_per_step):
        w_row = wid_ref[r]
        m_row = am_ref[r]
        valid = (m_row != 0) & (w_row != -1)
        prev = jnp.concatenate(
            [jnp.full((1, 1), -2, jnp.int32), w_row[:, : ll - 1]], axis=1
        )
        new = valid & ((prev < 0) | (w_row != prev))
        newf = new.astype(jnp.bfloat16)
        cums = jnp.dot(newf, lt, preferred_element_type=jnp.float32)
        seg = cums.astype(jnp.int32) - 1
        oneh = ((sid == seg) & valid).astype(jnp.bfloat16)
        counts = jnp.sum(oneh.astype(jnp.float32), axis=1, keepdims=True)
        pooled = jnp.dot(oneh, scores[r * ll : (r + 1) * ll].astype(jnp.bfloat16),
                         preferred_element_type=jnp.float32)
        pooled = pooled / jnp.maximum(counts, 1.0)
        s = pooled[:maxS, 0:1]
        t = lax.dot_general(
            e1, pooled, (((1,), (1,)), ((), ())),
            preferred_element_type=jnp.float32,
        )
        o_ref[r, :, :] = jnp.pad(
            s + t[:, maxS : maxS + maxT] + bc_ref[0],
            ((0, _OPAD - maxS), (0, _OPAD - maxT)),
        )


def _tc_forward(x, w_enc, b_enc, w2, wid, am, b_cls, maxS, maxT):
    bb, ll, hh = x.shape
    rr = 2
    body = functools.partial(_tc_body, maxS=maxS, maxT=maxT, rows_per_step=rr)
    return pl.pallas_call(
        body,
        grid=(bb // rr,),
        in_specs=[
            pl.BlockSpec((rr, ll, hh), lambda b: (b, 0, 0)),
            pl.BlockSpec((hh, hh), lambda b: (0, 0)),
            pl.BlockSpec((1, hh), lambda b: (0, 0)),
            pl.BlockSpec((2, hh), lambda b: (0, 0)),
            pl.BlockSpec((rr, 1, ll), lambda b: (b, 0, 0)),
            pl.BlockSpec((rr, 1, ll), lambda b: (b, 0, 0)),
            pl.BlockSpec(memory_space=pltpu.SMEM),
        ],
        out_specs=pl.BlockSpec((rr, _OPAD, _OPAD), lambda b: (b, 0, 0)),
        out_shape=jax.ShapeDtypeStruct((bb, _OPAD, _OPAD), jnp.float32),
    )(x, w_enc, b_enc, w2, wid, am, b_cls)


def kernel(input_ids, attention_mask, source_word_ids, target_word_ids,
           emb_table, W_enc, b_enc, W_cls, b_cls):
    bb, ll = input_ids.shape
    hh = emb_table.shape[1]
    tpw = 4
    maxS = (source_word_ids.shape[1] - tpw) // tpw
    maxT = (target_word_ids.shape[1] - tpw) // tpw

    x = _sc_gather(emb_table.astype(jnp.float32), input_ids.astype(jnp.int32))

    wid = jnp.concatenate(
        [source_word_ids, target_word_ids], axis=1
    ).astype(jnp.int32).reshape(bb, 1, ll)
    am = attention_mask.astype(jnp.int32).reshape(bb, 1, ll)

    w2 = W_cls.astype(jnp.float32)[:, 0].reshape(2, hh).astype(jnp.bfloat16)

    out = _tc_forward(
        x,
        W_enc.astype(jnp.bfloat16),
        b_enc.reshape(1, hh).astype(jnp.float32),
        w2,
        wid,
        am,
        b_cls.astype(jnp.float32),
        maxS,
        maxT,
    )
    return out[:, :maxS, :maxT]


# speedup vs baseline: 1.0096x; 1.0096x over previous
import functools

import jax
import jax.numpy as jnp
from jax import lax
from jax.experimental import pallas as pl
from jax.experimental.pallas import tpu as pltpu
from jax.experimental.pallas import tpu_sc as plsc

_NC, _NSUB = 2, 16
_NW = _NC * _NSUB
_NSEG = 128
_OPAD = 64


def _sc_gather(table, idx):
    bb, ll = idx.shape
    d = table.shape[1]
    rows_per_w = (bb * ll) // _NW
    w_per_row = ll // rows_per_w
    mesh = plsc.VectorSubcoreMesh(core_axis_name="c", subcore_axis_name="s")
    nstream = 2
    chunk = rows_per_w // nstream

    @functools.partial(
        pl.kernel,
        mesh=mesh,
        out_type=jax.ShapeDtypeStruct((bb, ll, d), jnp.float32),
        scratch_types=(
            [pltpu.VMEM((rows_per_w,), jnp.int32),
             pltpu.VMEM((rows_per_w, d), jnp.float32)]
            + [pltpu.SemaphoreType.DMA] * (2 * nstream)
        ),
    )
    def gather_kernel(table_hbm, idx_hbm, out_hbm, idx_v, rows_v, *sems):
        wid = lax.axis_index("s") * _NC + lax.axis_index("c")
        row = wid // w_per_row
        col = (wid % w_per_row) * rows_per_w
        pltpu.sync_copy(idx_hbm.at[row, pl.ds(col, rows_per_w)], idx_v)
        gathers = []
        for i in range(nstream):
            gathers.append(pltpu.async_copy(
                table_hbm.at[idx_v.at[pl.ds(i * chunk, chunk)]],
                rows_v.at[pl.ds(i * chunk, chunk)], sems[i]))
        writes = []
        for i in range(nstream):
            gathers[i].wait()
            writes.append(pltpu.async_copy(
                rows_v.at[pl.ds(i * chunk, chunk)],
                out_hbm.at[row, pl.ds(col + i * chunk, chunk)],
                sems[nstream + i]))
        for wcp in writes:
            wcp.wait()

    return gather_kernel(table, idx)


def _tc_body(x_ref, w_ref, b_ref, w2_ref, wid_ref, am_ref, bc_ref, o_ref,
             *, maxS, maxT, rows_per_step):
    ll = wid_ref.shape[-1]
    hh = x_ref.shape[-1]
    x = x_ref[...].reshape(rows_per_step * ll, hh).astype(jnp.bfloat16)
    h = jnp.tanh(
        jnp.dot(x, w_ref[...], preferred_element_type=jnp.float32) + b_ref[...]
    )
    scores = lax.dot_general(
        h.astype(jnp.bfloat16), w2_ref[...],
        (((1,), (1,)), ((), ())), preferred_element_type=jnp.float32,
    )
    lt = (
        lax.broadcasted_iota(jnp.int32, (ll, ll), 0)
        <= lax.broadcasted_iota(jnp.int32, (ll, ll), 1)
    ).astype(jnp.bfloat16)
    sid = lax.broadcasted_iota(jnp.int32, (_NSEG, ll), 0)
    e1 = (lax.broadcasted_iota(jnp.int32, (1, 2), 1) == 1).astype(jnp.float32)
    for r in range(rows_per_step):
        w_row = wid_ref[r]
        m_row = am_ref[r]
        valid = (m_row != 0) & (w_row != -1)
        prev = jnp.concatenate(
            [jnp.full((1, 1), -2, jnp.int32), w_row[:, : ll - 1]], axis=1
        )
        new = valid & ((prev < 0) | (w_row != prev))
        newf = new.astype(jnp.bfloat16)
        cums = jnp.dot(newf, lt, preferred_element_type=jnp.float32)
        seg = cums.astype(jnp.int32) - 1
        oneh = ((sid == seg) & valid).astype(jnp.bfloat16)
        counts = jnp.sum(oneh.astype(jnp.float32), axis=1, keepdims=True)
        pooled = jnp.dot(oneh, scores[r * ll : (r + 1) * ll].astype(jnp.bfloat16),
                         preferred_element_type=jnp.float32)
        pooled = pooled / jnp.maximum(counts, 1.0)
        s = pooled[:maxS, 0:1]
        t = lax.dot_general(
            e1, pooled, (((1,), (1,)), ((), ())),
            preferred_element_type=jnp.float32,
        )
        o_ref[r, :, :] = jnp.pad(
            s + t[:, maxS : maxS + maxT] + bc_ref[0],
            ((0, _OPAD - maxS), (0, _OPAD - maxT)),
        )


def _tc_forward(x, w_enc, b_enc, w2, wid, am, b_cls, maxS, maxT):
    bb, ll, hh = x.shape
    rr = 2
    body = functools.partial(_tc_body, maxS=maxS, maxT=maxT, rows_per_step=rr)
    return pl.pallas_call(
        body,
        grid=(bb // rr,),
        in_specs=[
            pl.BlockSpec((rr, ll, hh), lambda b: (b, 0, 0)),
            pl.BlockSpec((hh, hh), lambda b: (0, 0)),
            pl.BlockSpec((1, hh), lambda b: (0, 0)),
            pl.BlockSpec((2, hh), lambda b: (0, 0)),
            pl.BlockSpec((rr, 1, ll), lambda b: (b, 0, 0)),
            pl.BlockSpec((rr, 1, ll), lambda b: (b, 0, 0)),
            pl.BlockSpec(memory_space=pltpu.SMEM),
        ],
        out_specs=pl.BlockSpec((rr, _OPAD, _OPAD), lambda b: (b, 0, 0)),
        out_shape=jax.ShapeDtypeStruct((bb, _OPAD, _OPAD), jnp.float32),
    )(x, w_enc, b_enc, w2, wid, am, b_cls)


def kernel(input_ids, attention_mask, source_word_ids, target_word_ids,
           emb_table, W_enc, b_enc, W_cls, b_cls):
    bb, ll = input_ids.shape
    hh = emb_table.shape[1]
    tpw = 4
    maxS = (source_word_ids.shape[1] - tpw) // tpw
    maxT = (target_word_ids.shape[1] - tpw) // tpw

    x = _sc_gather(emb_table.astype(jnp.float32), input_ids.astype(jnp.int32))

    wid = jnp.concatenate(
        [source_word_ids, target_word_ids], axis=1
    ).astype(jnp.int32).reshape(bb, 1, ll)
    am = attention_mask.astype(jnp.int32).reshape(bb, 1, ll)

    w2 = W_cls.astype(jnp.float32)[:, 0].reshape(2, hh).astype(jnp.bfloat16)

    out = _tc_forward(
        x,
        W_enc.astype(jnp.bfloat16),
        b_enc.reshape(1, hh).astype(jnp.float32),
        w2,
        wid,
        am,
        b_cls.astype(jnp.float32),
        maxS,
        maxT,
    )
    return out[:, :maxS, :maxT]


# speedup vs baseline: 1.0273x; 1.0175x over previous
import functools

import jax
import jax.numpy as jnp
from jax import lax
from jax.experimental import pallas as pl
from jax.experimental.pallas import tpu as pltpu
from jax.experimental.pallas import tpu_sc as plsc

_NC, _NSUB = 2, 16
_NW = _NC * _NSUB
_NSEG = 128
_OPAD = 64


def _sc_gather(table, idx):
    bb, ll = idx.shape
    d = table.shape[1]
    rows_per_w = (bb * ll) // _NW
    w_per_row = ll // rows_per_w
    mesh = plsc.VectorSubcoreMesh(core_axis_name="c", subcore_axis_name="s")
    nstream = 2
    chunk = rows_per_w // nstream

    @functools.partial(
        pl.kernel,
        mesh=mesh,
        out_type=jax.ShapeDtypeStruct((bb, ll, d), jnp.float32),
        scratch_types=(
            [pltpu.VMEM((rows_per_w,), jnp.int32),
             pltpu.VMEM((rows_per_w, d), jnp.float32)]
            + [pltpu.SemaphoreType.DMA] * (2 * nstream)
        ),
    )
    def gather_kernel(table_hbm, idx_hbm, out_hbm, idx_v, rows_v, *sems):
        wid = lax.axis_index("s") * _NC + lax.axis_index("c")
        row = wid // w_per_row
        col = (wid % w_per_row) * rows_per_w
        pltpu.sync_copy(idx_hbm.at[row, pl.ds(col, rows_per_w)], idx_v)
        gathers = []
        for i in range(nstream):
            gathers.append(pltpu.async_copy(
                table_hbm.at[idx_v.at[pl.ds(i * chunk, chunk)]],
                rows_v.at[pl.ds(i * chunk, chunk)], sems[i]))
        writes = []
        for i in range(nstream):
            gathers[i].wait()
            writes.append(pltpu.async_copy(
                rows_v.at[pl.ds(i * chunk, chunk)],
                out_hbm.at[row, pl.ds(col + i * chunk, chunk)],
                sems[nstream + i]))
        for wcp in writes:
            wcp.wait()

    return gather_kernel(table, idx)


def _tc_body(xa_ref, xb_ref, w_ref, b_ref, w2_ref, wid_ref, am_ref, bc_ref,
             o_ref, *, maxS, maxT, rows_per_step):
    ll = wid_ref.shape[-1]
    hh = xa_ref.shape[-1]
    ll2 = ll // 2
    scores_h = []
    for x_ref in (xa_ref, xb_ref):
        x = x_ref[...].reshape(rows_per_step * ll2, hh).astype(jnp.bfloat16)
        h = jnp.tanh(
            jnp.dot(x, w_ref[...], preferred_element_type=jnp.float32)
            + b_ref[...]
        )
        scores_h.append(lax.dot_general(
            h.astype(jnp.bfloat16), w2_ref[...],
            (((1,), (1,)), ((), ())), preferred_element_type=jnp.float32,
        ))
    scores_a, scores_b = scores_h
    lt = (
        lax.broadcasted_iota(jnp.int32, (ll, ll), 0)
        <= lax.broadcasted_iota(jnp.int32, (ll, ll), 1)
    ).astype(jnp.bfloat16)
    sid = lax.broadcasted_iota(jnp.int32, (_NSEG, ll), 0)
    e1 = (lax.broadcasted_iota(jnp.int32, (1, 2), 1) == 1).astype(jnp.float32)
    for r in range(rows_per_step):
        w_row = wid_ref[r]
        m_row = am_ref[r]
        valid = (m_row != 0) & (w_row != -1)
        prev = jnp.concatenate(
            [jnp.full((1, 1), -2, jnp.int32), w_row[:, : ll - 1]], axis=1
        )
        new = valid & ((prev < 0) | (w_row != prev))
        newf = new.astype(jnp.bfloat16)
        cums = jnp.dot(newf, lt, preferred_element_type=jnp.float32)
        seg = cums.astype(jnp.int32) - 1
        oneh = ((sid == seg) & valid).astype(jnp.bfloat16)
        counts = jnp.sum(oneh.astype(jnp.float32), axis=1, keepdims=True)
        pooled = jnp.dot(
            oneh[:, :ll2],
            scores_a[r * ll2 : (r + 1) * ll2].astype(jnp.bfloat16),
            preferred_element_type=jnp.float32,
        ) + jnp.dot(
            oneh[:, ll2:],
            scores_b[r * ll2 : (r + 1) * ll2].astype(jnp.bfloat16),
            preferred_element_type=jnp.float32,
        )
        pooled = pooled / jnp.maximum(counts, 1.0)
        s = pooled[:maxS, 0:1]
        t = lax.dot_general(
            e1, pooled, (((1,), (1,)), ((), ())),
            preferred_element_type=jnp.float32,
        )
        o_ref[r, :, :] = jnp.pad(
            s + t[:, maxS : maxS + maxT] + bc_ref[0],
            ((0, _OPAD - maxS), (0, _OPAD - maxT)),
        )


def _tc_forward(x, w_enc, b_enc, w2, wid, am, b_cls, maxS, maxT):
    bb, ll, hh = x.shape
    rr = 2
    body = functools.partial(_tc_body, maxS=maxS, maxT=maxT, rows_per_step=rr)
    return pl.pallas_call(
        body,
        grid=(bb // rr,),
        in_specs=[
            pl.BlockSpec((rr, ll // 2, hh), lambda b: (b, 0, 0)),
            pl.BlockSpec((rr, ll // 2, hh), lambda b: (b, 1, 0)),
            pl.BlockSpec((hh, hh), lambda b: (0, 0)),
            pl.BlockSpec((1, hh), lambda b: (0, 0)),
            pl.BlockSpec((2, hh), lambda b: (0, 0)),
            pl.BlockSpec((rr, 1, ll), lambda b: (b, 0, 0)),
            pl.BlockSpec((rr, 1, ll), lambda b: (b, 0, 0)),
            pl.BlockSpec(memory_space=pltpu.SMEM),
        ],
        out_specs=pl.BlockSpec((rr, _OPAD, _OPAD), lambda b: (b, 0, 0)),
        out_shape=jax.ShapeDtypeStruct((bb, _OPAD, _OPAD), jnp.float32),
    )(x, x, w_enc, b_enc, w2, wid, am, b_cls)


def kernel(input_ids, attention_mask, source_word_ids, target_word_ids,
           emb_table, W_enc, b_enc, W_cls, b_cls):
    bb, ll = input_ids.shape
    hh = emb_table.shape[1]
    tpw = 4
    maxS = (source_word_ids.shape[1] - tpw) // tpw
    maxT = (target_word_ids.shape[1] - tpw) // tpw

    x = _sc_gather(emb_table.astype(jnp.float32), input_ids.astype(jnp.int32))

    wid = jnp.concatenate(
        [source_word_ids, target_word_ids], axis=1
    ).astype(jnp.int32).reshape(bb, 1, ll)
    am = attention_mask.astype(jnp.int32).reshape(bb, 1, ll)

    w2 = W_cls.astype(jnp.float32)[:, 0].reshape(2, hh).astype(jnp.bfloat16)

    out = _tc_forward(
        x,
        W_enc.astype(jnp.bfloat16),
        b_enc.reshape(1, hh).astype(jnp.float32),
        w2,
        wid,
        am,
        b_cls.astype(jnp.float32),
        maxS,
        maxT,
    )
    return out[:, :maxS, :maxT]
